# baseline (speedup 1.0000x reference)
import functools

import jax
import jax.numpy as jnp
from jax import lax
from jax.experimental import pallas as pl
from jax.experimental.pallas import tpu as pltpu
from jax.experimental.pallas import tpu_sc as plsc

NC = 2
NS = 16
L = 16
NW = NC * NS
B = 16384
K = 32
BPW = B // NW
CH = 128
NCH = BPW // CH
NG = CH // L


def _lfm_body(sampleU_h, sampleI_h, sampleR_h, alpha_h, betaU_h, betaI_h,
              gammaUT_h, gammaIT_h, out_h,
              idxU_v, idxI_v, rowsU_v, rowsI_v,
              bu_v, bi_v, r_v, alpha_v, acc_v, sem, gsem):
    c = lax.axis_index("c")
    s = lax.axis_index("s")
    wid = s * NC + c
    base = wid * BPW

    pltpu.sync_copy(sampleU_h.at[pl.ds(base, BPW)], idxU_v)
    pltpu.sync_copy(sampleI_h.at[pl.ds(base, BPW)], idxI_v)

    copies = []
    for j in range(NCH):
        dst = pl.ds(j * CH, CH)
        copies.append(pltpu.async_copy(
            betaU_h.at[idxU_v.at[dst]], bu_v.at[dst], sem))
        copies.append(pltpu.async_copy(
            betaI_h.at[idxI_v.at[dst]], bi_v.at[dst], sem))
    copies.append(pltpu.async_copy(
        sampleR_h.at[pl.ds(base, BPW)], r_v, sem))
    copies.append(pltpu.async_copy(alpha_h, alpha_v, sem))
    for cp in copies:
        cp.wait()

    def fire(j):
        pbase = (j % 2) * K * CH
        src = pl.ds(j * CH, CH)
        for k in range(K):
            pltpu.async_copy(gammaUT_h.at[k].at[idxU_v.at[src]],
                             rowsU_v.at[pl.ds(pbase + k * CH, CH)], gsem)
            pltpu.async_copy(gammaIT_h.at[k].at[idxI_v.at[src]],
                             rowsI_v.at[pl.ds(pbase + k * CH, CH)], gsem)

    def drain_round(par):
        pltpu.make_async_copy(
            sampleR_h.at[pl.ds(0, K * CH)],
            rowsU_v.at[pl.ds(par * K * CH, K * CH)], gsem).wait()
        pltpu.make_async_copy(
            sampleR_h.at[pl.ds(0, K * CH)],
            rowsI_v.at[pl.ds(par * K * CH, K * CH)], gsem).wait()

    def compute(j, acc):
        pbase = (j % 2) * K * CH
        for g in range(NG):
            dots = jnp.zeros((L,), jnp.float32)
            for k in range(K):
                off = pl.ds(pbase + k * CH + g * L, L)
                dots = dots + rowsU_v[off] * rowsI_v[off]
            soff = pl.ds(j * CH + g * L, L)
            diff = alpha_v[...] + bu_v[soff] + bi_v[soff] + dots - r_v[soff]
            acc = acc + diff * diff
        return acc

    fire(0)
    drain_round(0)

    def loop_body(j, acc):
        fire(j + 1)
        acc = compute(j, acc)
        drain_round((j + 1) % 2)
        return acc

    acc = lax.fori_loop(0, NCH - 1, loop_body, jnp.zeros((L,), jnp.float32))
    acc = compute(NCH - 1, acc)
    acc_v[...] = acc
    pltpu.sync_copy(acc_v, out_h.at[pl.ds(wid * L, L)])


@jax.jit
def _lfm(sampleU, sampleI, sampleR, alpha16, betaU, betaI, gammaUT, gammaIT):
    mesh = plsc.VectorSubcoreMesh(core_axis_name="c", subcore_axis_name="s")
    kern = functools.partial(
        pl.kernel, mesh=mesh,
        out_type=jax.ShapeDtypeStruct((NW * L,), jnp.float32),
        scratch_types=[
            pltpu.VMEM((BPW,), jnp.int32),
            pltpu.VMEM((BPW,), jnp.int32),
            pltpu.VMEM((2 * K * CH,), jnp.float32),
            pltpu.VMEM((2 * K * CH,), jnp.float32),
            pltpu.VMEM((BPW,), jnp.float32),
            pltpu.VMEM((BPW,), jnp.float32),
            pltpu.VMEM((BPW,), jnp.float32),
            pltpu.VMEM((L,), jnp.float32),
            pltpu.VMEM((L,), jnp.float32),
            pltpu.SemaphoreType.DMA,
            pltpu.SemaphoreType.DMA,
        ],
        compiler_params=pltpu.CompilerParams(
            needs_layout_passes=False, use_tc_tiling_on_sc=False),
    )(_lfm_body)
    return kern(sampleU, sampleI, sampleR, alpha16, betaU, betaI,
                gammaUT, gammaIT)


def kernel(sampleU, sampleI, sampleR, alpha, betaU, betaI, gammaU, gammaI):
    alpha16 = jnp.broadcast_to(alpha, (L,)).astype(jnp.float32)
    partials = _lfm(sampleU, sampleI, sampleR, alpha16,
                    betaU, betaI, gammaU.T, gammaI.T)
    return 0.5 * jnp.sum(partials) / sampleR.shape[0]
